# Initial kernel scaffold; baseline (speedup 1.0000x reference)
import functools

import jax
import jax.numpy as jnp
import numpy as np
from jax import lax
from jax.experimental import pallas as pl
from jax.experimental.pallas import tpu as pltpu

N = 10000
P = 320000
A = 64
EB = 32
EH = 32
AH = 64
SPH = 32
SP = 32
VEL = 32
C = 32
HEADS = 4
NRBF = 50
RMAX = 5.0
EPS = 1e-8
SCALE = 1.0

E_BLK = 3200


def _silu(x):
    return x * jax.nn.sigmoid(x)


def _edge1_body(gi_ref, gj_ref, W_ein_ref, b_ein_ref, W_eo1_ref, b_eo1_ref,
                W_eo2_ref, b_eo2_ref, W_att_ref, b_att_ref,
                h_edge_ref, att8_ref, dir8_ref):
    gi = gi_ref[...]
    gj = gj_ref[...]
    hi = gi[:, :A]
    hj = gj[:, :A]
    r = gj[:, A:A + 3] - gi[:, A:A + 3]
    d2 = jnp.sum(r * r, axis=-1, keepdims=True)
    d = jnp.sqrt(d2 + EPS)
    dirv = r / (d + EPS)
    h_cat = jnp.concatenate([hi, hj], axis=-1)
    mu = jnp.linspace(float(np.exp(-RMAX)), 1.0, NRBF).astype(jnp.float32)
    beta = ((2.0 / NRBF) * (1.0 - float(np.exp(-RMAX)))) ** -2
    rbf = jnp.exp(-beta * (jnp.exp(-d) - mu[None, :]) ** 2)
    filt = rbf * (h_cat @ W_ein_ref[...] + b_ein_ref[...])
    edge_in = jnp.concatenate([h_cat, filt, d / SCALE], axis=-1)
    h_edge = _silu(edge_in @ W_eo1_ref[...] + b_eo1_ref[...]) @ W_eo2_ref[...] + b_eo2_ref[...]
    att = h_edge @ W_att_ref[...] + b_att_ref[...]
    att = jnp.where(att > 0, att, 2.0 * (jnp.exp(att / 2.0) - 1.0))
    exp_att = jnp.exp(att)
    f_cut = 0.5 * (jnp.cos(jnp.pi * d / RMAX) + 1.0) * (d < RMAX).astype(jnp.float32)
    E = gi.shape[0]
    one = jnp.ones((E, 1), jnp.float32)
    zero3 = jnp.zeros((E, 3), jnp.float32)
    h_edge_ref[...] = h_edge
    att8_ref[...] = jnp.concatenate([exp_att, one, zero3], axis=-1)
    dir8_ref[...] = jnp.concatenate([dirv, f_cut, d, zero3], axis=-1)


def _edge1(gi, gj, W_ein, b_ein, W_eo1, b_eo1, W_eo2, b_eo2, W_att, b_att):
    grid = (P // E_BLK,)
    eb = lambda i: (i, 0)
    wb = lambda shape: pl.BlockSpec(shape, lambda i: (0,) * len(shape))
    return pl.pallas_call(
        _edge1_body,
        grid=grid,
        in_specs=[
            pl.BlockSpec((E_BLK, 80), eb),
            pl.BlockSpec((E_BLK, 80), eb),
            wb(W_ein.shape), wb(b_ein.shape), wb(W_eo1.shape), wb(b_eo1.shape),
            wb(W_eo2.shape), wb(b_eo2.shape), wb(W_att.shape), wb(b_att.shape),
        ],
        out_specs=[
            pl.BlockSpec((E_BLK, EB), eb),
            pl.BlockSpec((E_BLK, 8), eb),
            pl.BlockSpec((E_BLK, 8), eb),
        ],
        out_shape=[
            jax.ShapeDtypeStruct((P, EB), jnp.float32),
            jax.ShapeDtypeStruct((P, 8), jnp.float32),
            jax.ShapeDtypeStruct((P, 8), jnp.float32),
        ],
    )(gi, gj, W_ein, b_ein, W_eo1, b_eo1, W_eo2, b_eo2, W_att, b_att)


def _edge2_body(h_edge_ref, att8_ref, dir8_ref, den8_ref, Wx_ref,
                u_ref, comb_ref):
    h_edge = h_edge_ref[...]
    att8 = att8_ref[...]
    dir8 = dir8_ref[...]
    den = den8_ref[...]
    f_cut = dir8[:, 3:4]
    alpha = att8[:, 0:4] / (den[:, 0:4] + EPS) * f_cut
    t_pre = jnp.zeros_like(h_edge)
    for hh in range(HEADS):
        t_pre += alpha[:, hh:hh + 1] * (h_edge @ Wx_ref[hh * EB:(hh + 1) * EB, :])
    t = jnp.tanh(t_pre)
    u_ref[...] = jnp.concatenate(
        [h_edge * alpha[:, hh:hh + 1] for hh in range(HEADS)], axis=-1)
    comb_ref[...] = jnp.concatenate(
        [t * dir8[:, k:k + 1] for k in range(3)], axis=-1)


def _edge2(h_edge, att8, dir8, den8, Wx_r):
    grid = (P // E_BLK,)
    eb = lambda i: (i, 0)
    wb = lambda shape: pl.BlockSpec(shape, lambda i: (0,) * len(shape))
    return pl.pallas_call(
        _edge2_body,
        grid=grid,
        in_specs=[
            pl.BlockSpec((E_BLK, EB), eb),
            pl.BlockSpec((E_BLK, 8), eb),
            pl.BlockSpec((E_BLK, 8), eb),
            pl.BlockSpec((E_BLK, 8), eb),
            wb(Wx_r.shape),
        ],
        out_specs=[
            pl.BlockSpec((E_BLK, HEADS * EB), eb),
            pl.BlockSpec((E_BLK, 3 * C), eb),
        ],
        out_shape=[
            jax.ShapeDtypeStruct((P, HEADS * EB), jnp.float32),
            jax.ShapeDtypeStruct((P, 3 * C), jnp.float32),
        ],
    )(h_edge, att8, dir8, den8, Wx_r)


def _node_body(h_ref, x_ref, v_ref, uacc_ref, cacc_ref, seg8_ref,
               W_pn1_ref, b_pn1_ref, W_pn2_ref, b_pn2_ref,
               W_n1_ref, b_n1_ref, W_n2_ref, b_n2_ref,
               W_v1_ref, b_v1_ref, W_v2_ref, W_vmix_ref,
               h_out_ref, x_out_ref, v_out_ref):
    h = h_ref[...]
    x = x_ref[...]
    v = v_ref[...]
    uacc = uacc_ref[...]
    cacc = cacc_ref[...]
    cden = jnp.maximum(seg8_ref[...][:, 4:5], 1.0)
    cm = cacc / cden
    norm_sq = (cm[:, 0:C] ** 2 + cm[:, C:2 * C] ** 2 + cm[:, 2 * C:3 * C] ** 2)
    h_sp = _silu(_silu(norm_sq @ W_pn1_ref[...] + b_pn1_ref[...]) @ W_pn2_ref[...] + b_pn2_ref[...])
    node_in = jnp.concatenate([h, uacc, h_sp], axis=-1)
    h_up = h + _silu(_silu(node_in @ W_n1_ref[...] + b_n1_ref[...]) @ W_n2_ref[...] + b_n2_ref[...])
    dv = jnp.concatenate(
        [cm[:, k * C:(k + 1) * C] @ W_vmix_ref[...] for k in range(3)], axis=-1)
    scale_v = 2.0 * jax.nn.sigmoid(_silu(h_up @ W_v1_ref[...] + b_v1_ref[...]) @ W_v2_ref[...])
    v_up = scale_v * v + dv
    h_out_ref[...] = h_up
    x_out_ref[...] = x + v_up
    v_out_ref[...] = v_up


def _node(h, x, v, uacc, cacc, seg8, W_pn1, b_pn1, W_pn2, b_pn2,
          W_n1p, b_n1, W_n2, b_n2, W_v1, b_v1, W_v2, W_vmix):
    NB = 2000
    grid = (N // NB,)
    rb = lambda i: (i, 0)
    wb = lambda shape: pl.BlockSpec(shape, lambda i: (0,) * len(shape))
    return pl.pallas_call(
        _node_body,
        grid=grid,
        in_specs=[
            pl.BlockSpec((NB, A), rb),
            pl.BlockSpec((NB, 3), rb),
            pl.BlockSpec((NB, 3), rb),
            pl.BlockSpec((NB, HEADS * EB), rb),
            pl.BlockSpec((NB, 3 * C), rb),
            pl.BlockSpec((NB, 8), rb),
            wb(W_pn1.shape), wb(b_pn1.shape), wb(W_pn2.shape), wb(b_pn2.shape),
            wb(W_n1p.shape), wb(b_n1.shape), wb(W_n2.shape), wb(b_n2.shape),
            wb(W_v1.shape), wb(b_v1.shape), wb(W_v2.shape), wb(W_vmix.shape),
        ],
        out_specs=[
            pl.BlockSpec((NB, A), rb),
            pl.BlockSpec((NB, 3), rb),
            pl.BlockSpec((NB, 3), rb),
        ],
        out_shape=[
            jax.ShapeDtypeStruct((N, A), jnp.float32),
            jax.ShapeDtypeStruct((N, 3), jnp.float32),
            jax.ShapeDtypeStruct((N, 3), jnp.float32),
        ],
    )(h, x, v, uacc, cacc, seg8, W_pn1, b_pn1, W_pn2, b_pn2,
      W_n1p, b_n1, W_n2, b_n2, W_v1, b_v1, W_v2, W_vmix)


def kernel(h, x, v, pairlist, W_ein, b_ein, W_eo1, b_eo1, W_eo2, b_eo2,
           W_att, b_att, W_xmix, W_pn1, b_pn1, W_pn2, b_pn2, W_n1, b_n1,
           W_n2, b_n2, W_v1, b_v1, W_v2, W_vmix):
    idx_i = pairlist[0]
    idx_j = pairlist[1]

    T = jnp.concatenate([h, x, jnp.zeros((N, 13), jnp.float32)], axis=-1)

    wx_rows = (np.arange(EB)[None, :] * HEADS + np.arange(HEADS)[:, None])
    Wx_r = W_xmix[wx_rows.reshape(-1)]
    n1_perm = np.concatenate([
        np.arange(A),
        A + wx_rows.reshape(-1),
        A + HEADS * EB + np.arange(SP),
    ])
    W_n1p = W_n1[n1_perm]

    gi = T[idx_i]
    gj = T[idx_j]

    h_edge, att8, dir8 = _edge1(gi, gj, W_ein, b_ein, W_eo1, b_eo1,
                                W_eo2, b_eo2, W_att, b_att)

    seg8 = jax.ops.segment_sum(att8, idx_i, num_segments=N)
    den8 = seg8[idx_i]

    u, comb = _edge2(h_edge, att8, dir8, den8, Wx_r)

    uacc = jax.ops.segment_sum(u, idx_i, num_segments=N)
    cacc = jax.ops.segment_sum(comb, idx_i, num_segments=N)

    return _node(h, x, v, uacc, cacc, seg8, W_pn1, b_pn1, W_pn2, b_pn2,
                 W_n1p, b_n1, W_n2, b_n2, W_v1, b_v1, W_v2, W_vmix)


# speedup vs baseline: 6.7299x; 6.7299x over previous
import functools

import jax
import jax.numpy as jnp
import numpy as np
from jax import lax
from jax.experimental import pallas as pl
from jax.experimental.pallas import tpu as pltpu

N = 10000
P = 320000
A = 64
EB = 32
EH = 32
AH = 64
SPH = 32
SP = 32
VEL = 32
C = 32
HEADS = 4
NRBF = 50
RMAX = 5.0
EPS = 1e-8
SCALE = 1.0

E_BLK = 3200

_MU_NP = np.linspace(np.exp(-RMAX), 1.0, NRBF).astype(np.float32)[None, :]


def _silu(x):
    return x * jax.nn.sigmoid(x)


def _edge1_body(gi_ref, gj_ref, W_ein_ref, b_ein_ref, W_eo1_ref, b_eo1_ref,
                W_eo2_ref, b_eo2_ref, W_att_ref, b_att_ref,
                h_edge_ref, att8_ref, dir8_ref):
    gi = gi_ref[...]
    gj = gj_ref[...]
    hi = gi[:, :A]
    hj = gj[:, :A]
    r = gj[:, A:A + 3] - gi[:, A:A + 3]
    d2 = jnp.sum(r * r, axis=-1, keepdims=True)
    d = jnp.sqrt(d2 + EPS)
    dirv = r / (d + EPS)
    h_cat = jnp.concatenate([hi, hj], axis=-1)
    mu0 = float(np.exp(-RMAX))
    step = (1.0 - mu0) / (NRBF - 1)
    mu = mu0 + step * lax.broadcasted_iota(jnp.int32, (1, NRBF), 1).astype(jnp.float32)
    beta = ((2.0 / NRBF) * (1.0 - float(np.exp(-RMAX)))) ** -2
    rbf = jnp.exp(-beta * (jnp.exp(-d) - mu) ** 2)
    filt = rbf * (h_cat @ W_ein_ref[...] + b_ein_ref[...])
    edge_in = jnp.concatenate([h_cat, filt, d / SCALE], axis=-1)
    h_edge = _silu(edge_in @ W_eo1_ref[...] + b_eo1_ref[...]) @ W_eo2_ref[...] + b_eo2_ref[...]
    att = h_edge @ W_att_ref[...] + b_att_ref[...]
    att = jnp.where(att > 0, att, 2.0 * (jnp.exp(att / 2.0) - 1.0))
    exp_att = jnp.exp(att)
    f_cut = 0.5 * (jnp.cos(jnp.pi * d / RMAX) + 1.0) * (d < RMAX).astype(jnp.float32)
    E = gi.shape[0]
    one = jnp.ones((E, 1), jnp.float32)
    zero3 = jnp.zeros((E, 3), jnp.float32)
    h_edge_ref[...] = h_edge
    att8_ref[...] = jnp.concatenate([exp_att, one, zero3], axis=-1)
    dir8_ref[...] = jnp.concatenate([dirv, f_cut, d, zero3], axis=-1)


def _edge1(gi, gj, W_ein, b_ein, W_eo1, b_eo1, W_eo2, b_eo2, W_att, b_att):
    grid = (P // E_BLK,)
    eb = lambda i: (i, 0)
    wb = lambda shape: pl.BlockSpec(shape, lambda i: (0,) * len(shape))
    return pl.pallas_call(
        _edge1_body,
        grid=grid,
        in_specs=[
            pl.BlockSpec((E_BLK, 80), eb),
            pl.BlockSpec((E_BLK, 80), eb),
            wb(W_ein.shape), wb(b_ein.shape), wb(W_eo1.shape), wb(b_eo1.shape),
            wb(W_eo2.shape), wb(b_eo2.shape), wb(W_att.shape), wb(b_att.shape),
        ],
        out_specs=[
            pl.BlockSpec((E_BLK, EB), eb),
            pl.BlockSpec((E_BLK, 8), eb),
            pl.BlockSpec((E_BLK, 8), eb),
        ],
        out_shape=[
            jax.ShapeDtypeStruct((P, EB), jnp.float32),
            jax.ShapeDtypeStruct((P, 8), jnp.float32),
            jax.ShapeDtypeStruct((P, 8), jnp.float32),
        ],
    )(gi, gj, W_ein, b_ein, W_eo1, b_eo1, W_eo2, b_eo2, W_att, b_att)


def _edge2_body(h_edge_ref, att8_ref, dir8_ref, den8_ref, Wx_ref,
                u_ref, comb_ref):
    h_edge = h_edge_ref[...]
    att8 = att8_ref[...]
    dir8 = dir8_ref[...]
    den = den8_ref[...]
    f_cut = dir8[:, 3:4]
    alpha = att8[:, 0:4] / (den[:, 0:4] + EPS) * f_cut
    t_pre = jnp.zeros_like(h_edge)
    for hh in range(HEADS):
        t_pre += alpha[:, hh:hh + 1] * (h_edge @ Wx_ref[hh * EB:(hh + 1) * EB, :])
    t = jnp.tanh(t_pre)
    u_ref[...] = jnp.concatenate(
        [h_edge * alpha[:, hh:hh + 1] for hh in range(HEADS)], axis=-1)
    comb_ref[...] = jnp.concatenate(
        [t * dir8[:, k:k + 1] for k in range(3)], axis=-1)


def _edge2(h_edge, att8, dir8, den8, Wx_r):
    grid = (P // E_BLK,)
    eb = lambda i: (i, 0)
    wb = lambda shape: pl.BlockSpec(shape, lambda i: (0,) * len(shape))
    return pl.pallas_call(
        _edge2_body,
        grid=grid,
        in_specs=[
            pl.BlockSpec((E_BLK, EB), eb),
            pl.BlockSpec((E_BLK, 8), eb),
            pl.BlockSpec((E_BLK, 8), eb),
            pl.BlockSpec((E_BLK, 8), eb),
            wb(Wx_r.shape),
        ],
        out_specs=[
            pl.BlockSpec((E_BLK, HEADS * EB), eb),
            pl.BlockSpec((E_BLK, 3 * C), eb),
        ],
        out_shape=[
            jax.ShapeDtypeStruct((P, HEADS * EB), jnp.float32),
            jax.ShapeDtypeStruct((P, 3 * C), jnp.float32),
        ],
    )(h_edge, att8, dir8, den8, Wx_r)


def _node_body(h_ref, x_ref, v_ref, uacc_ref, cacc_ref, seg8_ref,
               W_pn1_ref, b_pn1_ref, W_pn2_ref, b_pn2_ref,
               W_n1_ref, b_n1_ref, W_n2_ref, b_n2_ref,
               W_v1_ref, b_v1_ref, W_v2_ref, W_vmix_ref,
               h_out_ref, x_out_ref, v_out_ref):
    h = h_ref[...]
    x = x_ref[...]
    v = v_ref[...]
    uacc = uacc_ref[...]
    cacc = cacc_ref[...]
    cden = jnp.maximum(seg8_ref[...][:, 4:5], 1.0)
    cm = cacc / cden
    norm_sq = (cm[:, 0:C] ** 2 + cm[:, C:2 * C] ** 2 + cm[:, 2 * C:3 * C] ** 2)
    h_sp = _silu(_silu(norm_sq @ W_pn1_ref[...] + b_pn1_ref[...]) @ W_pn2_ref[...] + b_pn2_ref[...])
    node_in = jnp.concatenate([h, uacc, h_sp], axis=-1)
    h_up = h + _silu(_silu(node_in @ W_n1_ref[...] + b_n1_ref[...]) @ W_n2_ref[...] + b_n2_ref[...])
    dv = jnp.concatenate(
        [cm[:, k * C:(k + 1) * C] @ W_vmix_ref[...] for k in range(3)], axis=-1)
    scale_v = 2.0 * jax.nn.sigmoid(_silu(h_up @ W_v1_ref[...] + b_v1_ref[...]) @ W_v2_ref[...])
    v_up = scale_v * v + dv
    h_out_ref[...] = h_up
    x_out_ref[...] = x + v_up
    v_out_ref[...] = v_up


def _node(h, x, v, uacc, cacc, seg8, W_pn1, b_pn1, W_pn2, b_pn2,
          W_n1p, b_n1, W_n2, b_n2, W_v1, b_v1, W_v2, W_vmix):
    NB = 2000
    grid = (N // NB,)
    rb = lambda i: (i, 0)
    wb = lambda shape: pl.BlockSpec(shape, lambda i: (0,) * len(shape))
    return pl.pallas_call(
        _node_body,
        grid=grid,
        in_specs=[
            pl.BlockSpec((NB, A), rb),
            pl.BlockSpec((NB, 3), rb),
            pl.BlockSpec((NB, 3), rb),
            pl.BlockSpec((NB, HEADS * EB), rb),
            pl.BlockSpec((NB, 3 * C), rb),
            pl.BlockSpec((NB, 8), rb),
            wb(W_pn1.shape), wb(b_pn1.shape), wb(W_pn2.shape), wb(b_pn2.shape),
            wb(W_n1p.shape), wb(b_n1.shape), wb(W_n2.shape), wb(b_n2.shape),
            wb(W_v1.shape), wb(b_v1.shape), wb(W_v2.shape), wb(W_vmix.shape),
        ],
        out_specs=[
            pl.BlockSpec((NB, A), rb),
            pl.BlockSpec((NB, 3), rb),
            pl.BlockSpec((NB, 3), rb),
        ],
        out_shape=[
            jax.ShapeDtypeStruct((N, A), jnp.float32),
            jax.ShapeDtypeStruct((N, 3), jnp.float32),
            jax.ShapeDtypeStruct((N, 3), jnp.float32),
        ],
    )(h, x, v, uacc, cacc, seg8, W_pn1, b_pn1, W_pn2, b_pn2,
      W_n1p, b_n1, W_n2, b_n2, W_v1, b_v1, W_v2, W_vmix)


def kernel(h, x, v, pairlist, W_ein, b_ein, W_eo1, b_eo1, W_eo2, b_eo2,
           W_att, b_att, W_xmix, W_pn1, b_pn1, W_pn2, b_pn2, W_n1, b_n1,
           W_n2, b_n2, W_v1, b_v1, W_v2, W_vmix):
    idx_i = pairlist[0]
    idx_j = pairlist[1]

    T = jnp.concatenate([h, x, jnp.zeros((N, 13), jnp.float32)], axis=-1)

    wx_rows = (np.arange(EB)[None, :] * HEADS + np.arange(HEADS)[:, None])
    Wx_r = W_xmix[wx_rows.reshape(-1)]
    n1_perm = np.concatenate([
        np.arange(A),
        A + wx_rows.reshape(-1),
        A + HEADS * EB + np.arange(SP),
    ])
    W_n1p = W_n1[n1_perm]

    gi = T[idx_i]
    gj = T[idx_j]

    h_edge, att8, dir8 = _edge1(gi, gj, W_ein, b_ein, W_eo1, b_eo1,
                                W_eo2, b_eo2, W_att, b_att)

    seg8 = jax.ops.segment_sum(att8, idx_i, num_segments=N)
    den8 = seg8[idx_i]

    u, comb = _edge2(h_edge, att8, dir8, den8, Wx_r)

    uacc = jax.ops.segment_sum(u, idx_i, num_segments=N)
    cacc = jax.ops.segment_sum(comb, idx_i, num_segments=N)

    return _node(h, x, v, uacc, cacc, seg8, W_pn1, b_pn1, W_pn2, b_pn2,
                 W_n1p, b_n1, W_n2, b_n2, W_v1, b_v1, W_v2, W_vmix)
